# baseline (speedup 1.0000x reference)
import functools

import jax
import jax.numpy as jnp
from jax import lax
from jax.experimental import pallas as pl
from jax.experimental.pallas import tpu as pltpu
from jax.experimental.pallas import tpu_sc as plsc

_NC = 2
_NS = 16
_NW = _NC * _NS
_L = 16

_GRID = 513
_SENTINEL = 30000.0

_SX = (0, -1, 1)
_SY = (0, 1, -1)

_TCB = 512


def _wrap(v):
    v = jnp.where(v < 0, v + _GRID, v)
    return jnp.where(v >= _GRID, v - _GRID, v)


def _vfull(v, dtype=jnp.float32):
    return jnp.full((_L,), v, dtype)


def _newton_sqrt(x):
    i = lax.bitcast_convert_type(x, jnp.int32)
    y = lax.bitcast_convert_type(
        lax.shift_right_logical(i, _vfull(1, jnp.int32))
        + _vfull(0x1FBD1DF5, jnp.int32),
        jnp.float32,
    )
    half = _vfull(0.5)
    y = half * (y + x / y)
    y = half * (y + x / y)
    return y


def _gauss_tc(bxr, byr, vx, vy, p4):
    grid, _, _ = bxr.shape
    n_pt = p4.shape[-1]

    def body_mxu(
        bx_ref, by_ref, vx_ref, vy_ref, p4_ref, out_ref, ptx_ref, qtx_ref
    ):
        px = vx_ref[0]
        py = vy_ref[0]
        p4v = p4_ref[0]
        valid = jnp.abs(p4v) > 0.1
        flag = jnp.where(valid, jnp.float32(0.0), jnp.float32(1.7e9))
        cnt = jnp.sum(jnp.where(valid, 1, 0))
        scale = jnp.where(cnt > 1, jnp.float32(0.01), jnp.float32(0.0))
        z = jnp.zeros_like(px)
        ptx_ref[0, 0] = px * px + flag
        ptx_ref[0, 1] = px
        ptx_ref[1, 0] = py * py
        ptx_ref[1, 1] = py
        for r in range(2, 8):
            ptx_ref[0, r] = z
            ptx_ref[1, r] = z

        bx = bx_ref[0, 0]
        by = by_ref[0, 0]
        nb = bx.shape[0]
        qx = [
            bx,
            jnp.where(bx == 0, _GRID - 1, bx - 1),
            jnp.where(bx >= _GRID - 1, 0, bx + 1),
        ]
        qy = [
            by,
            jnp.where(by >= _GRID - 1, 0, by + 1),
            jnp.where(by == 0, _GRID - 1, by - 1),
        ]
        qxf = [q.astype(jnp.float32) for q in qx]
        qyf = [q.astype(jnp.float32) for q in qy]
        oneb = jnp.ones((nb,), jnp.float32)
        zb = jnp.zeros((nb,), jnp.float32)
        for k in range(3):
            sl = pl.ds(k * nb, nb)
            qtx_ref[0, 0, sl] = oneb
            qtx_ref[1, 0, sl] = oneb
            qtx_ref[0, 1, sl] = jnp.float32(-2.0) * qxf[k]
            qtx_ref[1, 1, sl] = jnp.float32(-2.0) * qyf[k]
            for r in range(2, 8):
                qtx_ref[0, r, sl] = zb
                qtx_ref[1, r, sl] = zb

        chunk = 256
        dnums = (((0,), (0,)), ((), ()))
        accs = {j: jnp.full((nb,), 3e38, jnp.float32) for j in range(9)}
        for c in range(n_pt // chunk):
            csl = pl.ds(c * chunk, chunk)
            u = lax.dot_general(
                ptx_ref[0, :, csl], qtx_ref[0],
                dnums, preferred_element_type=jnp.float32,
            )
            v = lax.dot_general(
                ptx_ref[1, :, csl], qtx_ref[1],
                dnums, preferred_element_type=jnp.float32,
            )
            for ll in range(3):
                for kk in range(3):
                    j = 3 * ll + kk
                    s = (
                        u[:, kk * nb : (kk + 1) * nb]
                        + v[:, ll * nb : (ll + 1) * nb]
                    )
                    accs[j] = jnp.minimum(accs[j], jnp.min(s, axis=0))

        for ll in range(3):
            for kk in range(3):
                j = 3 * ll + kk
                qn = qxf[kk] * qxf[kk] + qyf[ll] * qyf[ll]
                out_ref[0, j, 0] = jnp.sqrt(accs[j] + qn) * scale

    return pl.pallas_call(
        body_mxu,
        grid=(grid,),
        in_specs=[
            pl.BlockSpec((1, 1, _TCB), lambda g: (g, 0, 0)),
            pl.BlockSpec((1, 1, _TCB), lambda g: (g, 0, 0)),
            pl.BlockSpec((1, n_pt), lambda g: (0, 0)),
            pl.BlockSpec((1, n_pt), lambda g: (0, 0)),
            pl.BlockSpec((1, n_pt), lambda g: (0, 0)),
        ],
        out_specs=pl.BlockSpec((1, 9, 1, _TCB), lambda g: (g, 0, 0, 0)),
        out_shape=jax.ShapeDtypeStruct((grid, 9, 1, _TCB), jnp.float32),
        scratch_shapes=[
            pltpu.VMEM((2, 8, n_pt), jnp.float32),
            pltpu.VMEM((2, 8, 3 * _TCB), jnp.float32),
        ],
    )(bxr, byr, vx, vy, p4)


def _gauss_sc(rax, ray, vx, vy, p4):
    n_ra = rax.shape[0]
    n_pt = vx.shape[0]
    nb_ra = n_ra // _NW
    mesh = plsc.VectorSubcoreMesh(core_axis_name="c", subcore_axis_name="s")

    @functools.partial(
        pl.kernel,
        mesh=mesh,
        out_type=[
            jax.ShapeDtypeStruct((n_ra * _L,), jnp.float32),
        ],
        scratch_types=[
            pltpu.VMEM((n_pt,), jnp.float32),
            pltpu.VMEM((n_pt,), jnp.float32),
            pltpu.VMEM((n_pt,), jnp.float32),
            pltpu.VMEM((n_pt,), jnp.float32),
            pltpu.SMEM((nb_ra,), jnp.int32),
            pltpu.SMEM((nb_ra,), jnp.int32),
            pltpu.VMEM((nb_ra * _L,), jnp.float32),
            pltpu.VMEM((nb_ra,), jnp.int32),
    ],
    )
    def k(
        rax_hbm,
        ray_hbm,
        vx_hbm,
        vy_hbm,
        p4_hbm,
        ra_out_hbm,
        px_v,
        py_v,
        pn_v,
        p4_v,
        bxr_v,
        byr_v,
        or_v,
        tmp_v,
    ):
        wid = lax.axis_index("s") * _NC + lax.axis_index("c")

        pltpu.sync_copy(vx_hbm, px_v)
        pltpu.sync_copy(vy_hbm, py_v)
        pltpu.sync_copy(p4_hbm, p4_v)
        for hbm, nb_c, smem in (
            (rax_hbm, nb_ra, bxr_v),
            (ray_hbm, nb_ra, byr_v),
        ):
            pltpu.sync_copy(
                hbm.at[pl.ds(wid * nb_c, nb_c)], tmp_v.at[pl.ds(0, nb_c)]
            )
            for g in range(nb_c // _L):
                vec = tmp_v[pl.ds(g * _L, _L)]
                for t in range(_L):
                    smem[g * _L + t] = vec[t]

        sent_v = _vfull(_SENTINEL)
        thresh_v = _vfull(0.1)
        ones_i = _vfull(1, jnp.int32)
        zero_i = _vfull(0, jnp.int32)
        lane = lax.iota(jnp.int32, _L)
        bfly = [lane ^ _vfull(k, jnp.int32) for k in (1, 2, 4, 8)]

        @plsc.parallel_loop(0, n_pt, _L, carry=zero_i)
        def _prep(i, cnt):
            sl = pl.ds(i, _L)
            valid = jnp.abs(p4_v[sl]) > thresh_v
            px = jnp.where(valid, px_v[sl], sent_v)
            py = jnp.where(valid, py_v[sl], sent_v)
            px_v[sl] = px
            py_v[sl] = py
            pn_v[sl] = px * px + py * py
            return cnt + jnp.where(valid, ones_i, zero_i)

        cnt = _prep
        for p in bfly:
            cnt = cnt + jnp.take(cnt, p)
        scale_v = jnp.where(cnt > ones_i, _vfull(0.01), _vfull(0.0))

        lane_is = [lane == _vfull(j, jnp.int32) for j in range(9)]

        def do_bases(nb, bx_v, by_v, out_v):
            def base_body(b, _):
                bx = bx_v[b]
                by = by_v[b]
                qx = [_wrap(bx + s).astype(jnp.float32) for s in _SX]
                qy = [_wrap(by + s).astype(jnp.float32) for s in _SY]
                m2x = [
                    jnp.broadcast_to(jnp.float32(-2.0) * q, (_L,)) for q in qx
                ]
                m2y = [
                    jnp.broadcast_to(jnp.float32(-2.0) * q, (_L,)) for q in qy
                ]
                init = tuple(_vfull(3e38) for _ in range(9))

                @plsc.parallel_loop(0, n_pt, _L, unroll=4, carry=init)
                def accs(i, acc):
                    sl = pl.ds(i, _L)
                    pxc = px_v[sl]
                    pyc = py_v[sl]
                    pnc = pn_v[sl]
                    u = [pnc + m2x[kk] * pxc for kk in range(3)]
                    w = [m2y[ll] * pyc for ll in range(3)]
                    return tuple(
                        jnp.minimum(acc[3 * ll + kk], u[kk] + w[ll])
                        for ll in range(3)
                        for kk in range(3)
                    )

                r = jnp.zeros((_L,), jnp.float32)
                for ll in range(3):
                    for kk in range(3):
                        j = 3 * ll + kk
                        qn = qx[kk] * qx[kk] + qy[ll] * qy[ll]
                        qn_v = jnp.broadcast_to(qn, (_L,))
                        m = accs[j]
                        for p in bfly:
                            m = jnp.minimum(m, jnp.take(m, p))
                        r = jnp.where(lane_is[j], m + qn_v, r)
                out_v[pl.ds(b * _L, _L)] = r
                return 0

            lax.fori_loop(0, nb, base_body, 0)

            @plsc.parallel_loop(0, nb * _L, _L)
            def _fin(i):
                sl = pl.ds(i, _L)
                out_v[sl] = _newton_sqrt(out_v[sl]) * scale_v

        do_bases(nb_ra, bxr_v, byr_v, or_v)

        pltpu.sync_copy(
            or_v, ra_out_hbm.at[pl.ds(wid * nb_ra * _L, nb_ra * _L)]
        )

    return k(rax, ray, vx, vy, p4)


def kernel(li_bev_coors, ra_bev_coors, ra_points, ra_voxel_coords):
    lidar_out = []
    radar_out = []
    B = ra_points.shape[0]
    for b in range(B):
        li = li_bev_coors[b].astype(jnp.int32)
        ra = ra_bev_coors[b].astype(jnp.int32)
        n_li = li.shape[0]
        n_ra = ra.shape[0]
        n_sc = n_ra // 2
        p4 = ra_points[b, :, 4].astype(jnp.float32)
        vx = ra_voxel_coords[b, :, 1].astype(jnp.float32)
        vy = ra_voxel_coords[b, :, 2].astype(jnp.float32)
        (sc_flat,) = _gauss_sc(
            ra[n_sc:, 0], ra[n_sc:, 1], vx, vy, p4
        )
        tc_bx = jnp.concatenate([li[:, 0], ra[:n_sc, 0]])
        tc_by = jnp.concatenate([li[:, 1], ra[:n_sc, 1]])
        n_tc = n_li + n_sc
        tc_out = _gauss_tc(
            tc_bx.reshape(n_tc // _TCB, 1, _TCB),
            tc_by.reshape(n_tc // _TCB, 1, _TCB),
            vx.reshape(1, -1),
            vy.reshape(1, -1),
            p4.reshape(1, -1),
        )
        tc_flat = (
            tc_out.reshape(n_tc // _TCB, 9, _TCB)
            .transpose(0, 2, 1)
            .reshape(n_tc, 9)
        )
        lidar_out.append(tc_flat[:n_li])
        radar_out.append(
            jnp.concatenate(
                [tc_flat[n_li:], sc_flat.reshape(n_sc, 16)[:, :9]]
            )
        )
    return (tuple(lidar_out), tuple(radar_out))


# speedup vs baseline: 1.0930x; 1.0930x over previous
import functools

import jax
import jax.numpy as jnp
from jax import lax
from jax.experimental import pallas as pl
from jax.experimental.pallas import tpu as pltpu
from jax.experimental.pallas import tpu_sc as plsc

_NC = 2
_NS = 16
_NW = _NC * _NS
_L = 16

_GRID = 513
_SENTINEL = 30000.0

_SX = (0, -1, 1)
_SY = (0, 1, -1)


def _wrap(v):
    v = jnp.where(v < 0, v + _GRID, v)
    return jnp.where(v >= _GRID, v - _GRID, v)


def _vfull(v, dtype=jnp.float32):
    return jnp.full((_L,), v, dtype)


def _newton_sqrt(x):
    i = lax.bitcast_convert_type(x, jnp.int32)
    y = lax.bitcast_convert_type(
        lax.shift_right_logical(i, _vfull(1, jnp.int32))
        + _vfull(0x1FBD1DF5, jnp.int32),
        jnp.float32,
    )
    half = _vfull(0.5)
    y = half * (y + x / y)
    y = half * (y + x / y)
    return y


def _gauss_tc(bxr, byr, vx, vy, p4):
    grid, _, _ = bxr.shape
    n_pt = p4.shape[-1]

    def body_mxu(
        bx_ref, by_ref, vx_ref, vy_ref, p4_ref, out_ref, ptx_ref, qtx_ref
    ):
        px = vx_ref[0]
        py = vy_ref[0]
        p4v = p4_ref[0]
        valid = jnp.abs(p4v) > 0.1
        flag = jnp.where(valid, jnp.float32(0.0), jnp.float32(1.7e9))
        cnt = jnp.sum(jnp.where(valid, 1, 0))
        scale = jnp.where(cnt > 1, jnp.float32(0.01), jnp.float32(0.0))
        z = jnp.zeros_like(px)
        ptx_ref[0, 0] = px * px + flag
        ptx_ref[0, 1] = px
        ptx_ref[1, 0] = py * py
        ptx_ref[1, 1] = py
        for r in range(2, 8):
            ptx_ref[0, r] = z
            ptx_ref[1, r] = z

        bx = bx_ref[0, 0]
        by = by_ref[0, 0]
        nb = bx.shape[0]
        qx = [
            bx,
            jnp.where(bx == 0, _GRID - 1, bx - 1),
            jnp.where(bx >= _GRID - 1, 0, bx + 1),
        ]
        qy = [
            by,
            jnp.where(by >= _GRID - 1, 0, by + 1),
            jnp.where(by == 0, _GRID - 1, by - 1),
        ]
        qxf = [q.astype(jnp.float32) for q in qx]
        qyf = [q.astype(jnp.float32) for q in qy]
        oneb = jnp.ones((nb,), jnp.float32)
        zb = jnp.zeros((nb,), jnp.float32)
        for k in range(3):
            sl = pl.ds(k * nb, nb)
            qtx_ref[0, 0, sl] = oneb
            qtx_ref[1, 0, sl] = oneb
            qtx_ref[0, 1, sl] = jnp.float32(-2.0) * qxf[k]
            qtx_ref[1, 1, sl] = jnp.float32(-2.0) * qyf[k]
            for r in range(2, 8):
                qtx_ref[0, r, sl] = zb
                qtx_ref[1, r, sl] = zb

        chunk = 256
        dnums = (((0,), (0,)), ((), ()))
        accs = {j: jnp.full((nb,), 3e38, jnp.float32) for j in range(9)}
        for c in range(n_pt // chunk):
            csl = pl.ds(c * chunk, chunk)
            u = lax.dot_general(
                ptx_ref[0, :, csl], qtx_ref[0],
                dnums, preferred_element_type=jnp.float32,
            )
            v = lax.dot_general(
                ptx_ref[1, :, csl], qtx_ref[1],
                dnums, preferred_element_type=jnp.float32,
            )
            for ll in range(3):
                for kk in range(3):
                    j = 3 * ll + kk
                    s = (
                        u[:, kk * nb : (kk + 1) * nb]
                        + v[:, ll * nb : (ll + 1) * nb]
                    )
                    accs[j] = jnp.minimum(accs[j], jnp.min(s, axis=0))

        for ll in range(3):
            for kk in range(3):
                j = 3 * ll + kk
                qn = qxf[kk] * qxf[kk] + qyf[ll] * qyf[ll]
                out_ref[0, j, 0] = jnp.sqrt(accs[j] + qn) * scale

    return pl.pallas_call(
        body_mxu,
        grid=(grid,),
        in_specs=[
            pl.BlockSpec((1, 1, 1024), lambda g: (g, 0, 0)),
            pl.BlockSpec((1, 1, 1024), lambda g: (g, 0, 0)),
            pl.BlockSpec((1, n_pt), lambda g: (0, 0)),
            pl.BlockSpec((1, n_pt), lambda g: (0, 0)),
            pl.BlockSpec((1, n_pt), lambda g: (0, 0)),
        ],
        out_specs=pl.BlockSpec((1, 9, 1, 1024), lambda g: (g, 0, 0, 0)),
        out_shape=jax.ShapeDtypeStruct((grid, 9, 1, 1024), jnp.float32),
        scratch_shapes=[
            pltpu.VMEM((2, 8, n_pt), jnp.float32),
            pltpu.VMEM((2, 8, 3 * 1024), jnp.float32),
        ],
    )(bxr, byr, vx, vy, p4)


def _gauss_sc(rax, ray, vx, vy, p4):
    n_ra = rax.shape[0]
    n_pt = vx.shape[0]
    nb_ra = n_ra // _NW
    mesh = plsc.VectorSubcoreMesh(core_axis_name="c", subcore_axis_name="s")

    @functools.partial(
        pl.kernel,
        mesh=mesh,
        out_type=[
            jax.ShapeDtypeStruct((n_ra * _L,), jnp.float32),
        ],
        scratch_types=[
            pltpu.VMEM((n_pt,), jnp.float32),
            pltpu.VMEM((n_pt,), jnp.float32),
            pltpu.VMEM((n_pt,), jnp.float32),
            pltpu.VMEM((n_pt,), jnp.float32),
            pltpu.SMEM((nb_ra,), jnp.int32),
            pltpu.SMEM((nb_ra,), jnp.int32),
            pltpu.VMEM((nb_ra * _L,), jnp.float32),
            pltpu.VMEM((nb_ra,), jnp.int32),
    ],
    )
    def k(
        rax_hbm,
        ray_hbm,
        vx_hbm,
        vy_hbm,
        p4_hbm,
        ra_out_hbm,
        px_v,
        py_v,
        pn_v,
        p4_v,
        bxr_v,
        byr_v,
        or_v,
        tmp_v,
    ):
        wid = lax.axis_index("s") * _NC + lax.axis_index("c")

        pltpu.sync_copy(vx_hbm, px_v)
        pltpu.sync_copy(vy_hbm, py_v)
        pltpu.sync_copy(p4_hbm, p4_v)
        for hbm, nb_c, smem in (
            (rax_hbm, nb_ra, bxr_v),
            (ray_hbm, nb_ra, byr_v),
        ):
            pltpu.sync_copy(
                hbm.at[pl.ds(wid * nb_c, nb_c)], tmp_v.at[pl.ds(0, nb_c)]
            )
            for g in range(nb_c // _L):
                vec = tmp_v[pl.ds(g * _L, _L)]
                for t in range(_L):
                    smem[g * _L + t] = vec[t]

        sent_v = _vfull(_SENTINEL)
        thresh_v = _vfull(0.1)
        ones_i = _vfull(1, jnp.int32)
        zero_i = _vfull(0, jnp.int32)
        lane = lax.iota(jnp.int32, _L)
        bfly = [lane ^ _vfull(k, jnp.int32) for k in (1, 2, 4, 8)]

        @plsc.parallel_loop(0, n_pt, _L, carry=zero_i)
        def _prep(i, cnt):
            sl = pl.ds(i, _L)
            valid = jnp.abs(p4_v[sl]) > thresh_v
            px = jnp.where(valid, px_v[sl], sent_v)
            py = jnp.where(valid, py_v[sl], sent_v)
            px_v[sl] = px
            py_v[sl] = py
            pn_v[sl] = px * px + py * py
            return cnt + jnp.where(valid, ones_i, zero_i)

        cnt = _prep
        for p in bfly:
            cnt = cnt + jnp.take(cnt, p)
        scale_v = jnp.where(cnt > ones_i, _vfull(0.01), _vfull(0.0))

        lane_is = [lane == _vfull(j, jnp.int32) for j in range(9)]

        def do_bases(nb, bx_v, by_v, out_v):
            def base_body(b, _):
                bx = bx_v[b]
                by = by_v[b]
                qx = [_wrap(bx + s).astype(jnp.float32) for s in _SX]
                qy = [_wrap(by + s).astype(jnp.float32) for s in _SY]
                m2x = [
                    jnp.broadcast_to(jnp.float32(-2.0) * q, (_L,)) for q in qx
                ]
                m2y = [
                    jnp.broadcast_to(jnp.float32(-2.0) * q, (_L,)) for q in qy
                ]
                init = tuple(_vfull(3e38) for _ in range(9))

                @plsc.parallel_loop(0, n_pt, _L, unroll=4, carry=init)
                def accs(i, acc):
                    sl = pl.ds(i, _L)
                    pxc = px_v[sl]
                    pyc = py_v[sl]
                    pnc = pn_v[sl]
                    u = [pnc + m2x[kk] * pxc for kk in range(3)]
                    w = [m2y[ll] * pyc for ll in range(3)]
                    return tuple(
                        jnp.minimum(acc[3 * ll + kk], u[kk] + w[ll])
                        for ll in range(3)
                        for kk in range(3)
                    )

                r = jnp.zeros((_L,), jnp.float32)
                for ll in range(3):
                    for kk in range(3):
                        j = 3 * ll + kk
                        qn = qx[kk] * qx[kk] + qy[ll] * qy[ll]
                        qn_v = jnp.broadcast_to(qn, (_L,))
                        m = accs[j]
                        for p in bfly:
                            m = jnp.minimum(m, jnp.take(m, p))
                        r = jnp.where(lane_is[j], m + qn_v, r)
                out_v[pl.ds(b * _L, _L)] = r
                return 0

            lax.fori_loop(0, nb, base_body, 0)

            @plsc.parallel_loop(0, nb * _L, _L)
            def _fin(i):
                sl = pl.ds(i, _L)
                out_v[sl] = _newton_sqrt(out_v[sl]) * scale_v

        do_bases(nb_ra, bxr_v, byr_v, or_v)

        pltpu.sync_copy(
            or_v, ra_out_hbm.at[pl.ds(wid * nb_ra * _L, nb_ra * _L)]
        )

    return k(rax, ray, vx, vy, p4)


def kernel(li_bev_coors, ra_bev_coors, ra_points, ra_voxel_coords):
    lidar_out = []
    radar_out = []
    B = ra_points.shape[0]
    for b in range(B):
        li = li_bev_coors[b].astype(jnp.int32)
        ra = ra_bev_coors[b].astype(jnp.int32)
        n_li = li.shape[0]
        n_ra = ra.shape[0]
        p4 = ra_points[b, :, 4].astype(jnp.float32)
        vx = ra_voxel_coords[b, :, 1].astype(jnp.float32)
        vy = ra_voxel_coords[b, :, 2].astype(jnp.float32)
        (ra_flat,) = _gauss_sc(ra[:, 0], ra[:, 1], vx, vy, p4)
        li_tc = _gauss_tc(
            li[:, 0].reshape(n_li // 1024, 1, 1024),
            li[:, 1].reshape(n_li // 1024, 1, 1024),
            vx.reshape(1, -1),
            vy.reshape(1, -1),
            p4.reshape(1, -1),
        )
        lidar_out.append(
            li_tc.reshape(n_li // 1024, 9, 1024)
            .transpose(0, 2, 1)
            .reshape(n_li, 9)
        )
        radar_out.append(ra_flat.reshape(n_ra, 16)[:, :9])
    return (tuple(lidar_out), tuple(radar_out))
